# baseline (speedup 1.0000x reference)
import functools

import jax
import jax.numpy as jnp
from jax import lax
from jax.experimental import pallas as pl
from jax.experimental.pallas import tpu as pltpu
from jax.experimental.pallas import tpu_sc as plsc

NC = 2
NS = 16
LANES = 16
CHUNK = 48
NBUF = 2


def _matmul_kernel(x_ref, w_ref, o_ref):
    o_ref[...] = lax.dot_general(
        x_ref[...], w_ref[...], (((1,), (1,)), ((), ())),
        preferred_element_type=jnp.float32)


def _build_edge_kernel(n, d, etot, ept):
    nchunks = ept // CHUNK
    rows_per_tile = ((n + NS * 128 - 1) // (NS * 128)) * 128
    npad = rows_per_tile * NS
    zreps = rows_per_tile // CHUNK
    ngroups = CHUNK // LANES

    mesh = plsc.VectorSubcoreMesh(
        core_axis_name="c", subcore_axis_name="s",
        num_cores=NC, num_subcores=NS)

    @functools.partial(
        pl.kernel,
        out_type=(
            jax.ShapeDtypeStruct((NC, npad, d), jnp.float32),
            jax.ShapeDtypeStruct((NC * NS, 1, n), jnp.float32),
        ),
        mesh=mesh,
        compiler_params=pltpu.CompilerParams(needs_layout_passes=False),
        scratch_types=(
            [pltpu.VMEM((CHUNK,), jnp.int32) for _ in range(4)]
            + [pltpu.VMEM((CHUNK,), jnp.int32) for _ in range(4)]
            + [pltpu.VMEM((CHUNK, d), jnp.float32) for _ in range(2)]
            + [pltpu.VMEM((CHUNK, d), jnp.float32) for _ in range(2)]
            + [pltpu.VMEM((CHUNK, d), jnp.float32) for _ in range(2)]
            + [
                pltpu.VMEM((n,), jnp.float32),
                pltpu.VMEM_SHARED((npad, d), jnp.float32),
                pltpu.SemaphoreType.DMA,
                pltpu.SemaphoreType.DMA,
                pltpu.SemaphoreType.DMA,
                pltpu.SemaphoreType.DMA,
                pltpu.SemaphoreType.DMA,
                pltpu.SemaphoreType.DMA,
                pltpu.SemaphoreType.DMA,
                pltpu.SemaphoreType.DMA,
            ]
        ),
    )
    def edge_kernel(emb_hbm, h_hbm, src_hbm, dst_hbm, po_hbm, den_hbm,
                    is0, is1, is2, is3, id0, id1, id2, id3,
                    es0, es1, ed0, ed1, hs0, hs1, den_t, acc_sh,
                    gsem0, gsem1, isem0, isem1, isem2, isem3, ssem0, ssem1):
        idx_s = [is0, is1, is2, is3]
        idx_d = [id0, id1, id2, id3]
        es2 = [es0, es1]
        ed2 = [ed0, ed1]
        hs2 = [hs0, hs1]
        gsem = [gsem0, gsem1]
        isem = [isem0, isem1, isem2, isem3]
        ssem = [ssem0, ssem1]
        cid = lax.axis_index("c")
        sid = lax.axis_index("s")
        wid = cid * NS + sid
        iota = lax.broadcasted_iota(jnp.int32, (LANES,), 0)
        zeros16 = jnp.zeros((LANES,), jnp.float32)
        ebase = wid * ept

        def zden(i, carry):
            den_t[pl.ds(i * LANES, LANES)] = zeros16
            return carry
        lax.fori_loop(0, n // LANES, zden, 0)

        def zz(i, carry):
            for j in range(d // LANES):
                hs0[i, pl.ds(j * LANES, LANES)] = zeros16
            return carry
        lax.fori_loop(0, CHUNK, zz, 0)
        row0 = sid * rows_per_tile
        left = rows_per_tile
        while left > 0:
            rr = min(left, CHUNK)
            pltpu.sync_copy(hs0.at[pl.ds(0, rr)],
                            acc_sh.at[pl.ds(row0 + rows_per_tile - left, rr)])
            left -= rr

        def fetch_idx(k, b4):
            base = ebase + k * CHUNK
            pltpu.sync_copy(src_hbm.at[pl.ds(base, CHUNK)], idx_s[b4])
            pltpu.sync_copy(dst_hbm.at[pl.ds(base, CHUNK)], idx_d[b4])

        def fetch_idx_async(k, b4):
            base = ebase + k * CHUNK
            pltpu.async_copy(src_hbm.at[pl.ds(base, CHUNK)], idx_s[b4],
                             isem[b4])
            pltpu.async_copy(dst_hbm.at[pl.ds(base, CHUNK)], idx_d[b4],
                             isem[b4])

        def wait_idx(b4):
            for _ in range(2):
                pltpu.make_async_copy(src_hbm.at[pl.ds(0, CHUNK)],
                                      idx_s[b4], isem[b4]).wait()

        def issue_embs(b4, b2):
            pltpu.async_copy(emb_hbm.at[idx_s[b4]], es2[b2], gsem[b2])
            pltpu.async_copy(emb_hbm.at[idx_d[b4]], ed2[b2], gsem[b2])

        def issue_h(b4, b2):
            pltpu.async_copy(h_hbm.at[idx_s[b4]], hs2[b2], gsem[b2])

        def drain(b2, m=3):
            for _ in range(m):
                pltpu.make_async_copy(emb_hbm.at[idx_s[0]], es2[b2],
                                      gsem[b2]).wait()

        def wait_scatter(b2):
            pltpu.make_async_copy(hs2[b2], acc_sh.at[pl.ds(0, CHUNK)],
                                  ssem[b2]).wait()

        for k in (0, 1):
            fetch_idx(k, k)
            issue_embs(k, k)
        issue_h(0, 0)
        fetch_idx_async(2, 2)
        fetch_idx_async(3, 3)
        plsc.subcore_barrier()

        def chunk_body(i, carry):
            for b4 in range(4):
                b2 = b4 % 2
                k = 4 * i + b4
                base = ebase + k * CHUNK
                es, ed, hs = es2[b2], ed2[b2], hs2[b2]
                drain(b2)

                for g in range(ngroups):
                    def edge_body(kk, ewg):
                        e = g * LANES + kk
                        acc = zeros16
                        for j in range(d // LANES):
                            sv = es[e, pl.ds(j * LANES, LANES)]
                            dv = ed[e, pl.ds(j * LANES, LANES)]
                            acc = acc + (sv - dv) * dv
                        av = acc
                        for sh in (8, 4, 2, 1):
                            av = av + jnp.take(av, iota ^ sh)
                        valid = jnp.full((LANES,), base + e, jnp.int32) < etot
                        ew = jnp.where(valid, jnp.exp(av), 0.0)
                        for j in range(d // LANES):
                            sl = pl.ds(j * LANES, LANES)
                            hs[e, sl] = hs[e, sl] * ew
                        return jnp.where(iota == kk, ew, ewg)
                    ewg = lax.fori_loop(0, LANES, edge_body, zeros16)
                    dd_idx = idx_d[b4][pl.ds(g * LANES, LANES)]
                    plsc.addupdate_scatter(den_t, [dd_idx], ewg)

                wait_idx((b4 + 2) % 4)
                issue_embs((b4 + 2) % 4, b2)

                pltpu.async_copy(hs, acc_sh.at[idx_d[b4]], ssem[b2],
                                 add=True)
                if b4 == 0:
                    @pl.when(i > 0)
                    def _():
                        wait_scatter(1 - b2)
                else:
                    wait_scatter(1 - b2)
                issue_h((b4 + 1) % 4, 1 - b2)
                fetch_idx_async(jnp.minimum(k + 4, nchunks - 1), b4)
            return carry
        lax.fori_loop(0, nchunks // 4, chunk_body, 0)
        drain(0, 3)
        drain(1, 2)
        wait_scatter(1)
        wait_idx(2)
        wait_idx(3)
        plsc.subcore_barrier()

        pltpu.sync_copy(acc_sh.at[pl.ds(row0, rows_per_tile)],
                        po_hbm.at[cid, pl.ds(row0, rows_per_tile)])
        pltpu.sync_copy(den_t, den_hbm.at[wid, 0])

    return edge_kernel


def _agg_kernel(po_ref, den_ref, bias_ref, agg_ref, sum_ref, sq_ref):
    i = pl.program_id(0)
    densum = jnp.sum(den_ref[...], axis=1, keepdims=True)
    agg = (po_ref[0] + po_ref[1]) / (densum + 1e-16) + bias_ref[...]
    agg_ref[...] = agg

    @pl.when(i == 0)
    def _():
        sum_ref[...] = jnp.zeros_like(sum_ref)
        sq_ref[...] = jnp.zeros_like(sq_ref)
    sum_ref[...] += jnp.sum(agg, axis=0, keepdims=True)
    sq_ref[...] += jnp.sum(agg * agg, axis=0, keepdims=True)


def _bn_kernel(n, agg_ref, sum_ref, sq_ref, bnw_ref, bnb_ref, o_ref):
    mean = sum_ref[...] / n
    var = sq_ref[...] / n - mean * mean
    y = (agg_ref[...] - mean) * lax.rsqrt(var + 1e-5) * bnw_ref[...]
    o_ref[...] = jnp.maximum(y + bnb_ref[...], 0.0)


def kernel(x, embedding, W, bias, bn_weight, bn_bias, edge_index):
    n, d_in = x.shape
    d = embedding.shape[1]
    e = edge_index.shape[1]
    etot = e + n
    ntiles = NC * NS
    step = 4 * CHUNK
    ept = ((etot + ntiles * step - 1) // (ntiles * step)) * step
    epad = ept * ntiles

    loop = jnp.arange(n, dtype=jnp.int32)
    src = jnp.concatenate([edge_index[0].astype(jnp.int32), loop,
                           jnp.zeros((epad - etot,), jnp.int32)])
    dst = jnp.concatenate([edge_index[1].astype(jnp.int32), loop,
                           jnp.zeros((epad - etot,), jnp.int32)])

    rblk = 2000
    nblocks = n // rblk
    h = pl.pallas_call(
        _matmul_kernel,
        grid=(nblocks,),
        in_specs=[
            pl.BlockSpec((rblk, d_in), lambda i: (i, 0)),
            pl.BlockSpec((d, d_in), lambda i: (0, 0)),
        ],
        out_specs=pl.BlockSpec((rblk, d), lambda i: (i, 0)),
        out_shape=jax.ShapeDtypeStruct((n, d), jnp.float32),
    )(x, W)

    edge_kernel = _build_edge_kernel(n, d, etot, ept)
    po, denp = edge_kernel(embedding, h, src, dst)

    agg, colsum, colsq = pl.pallas_call(
        _agg_kernel,
        grid=(nblocks,),
        in_specs=[
            pl.BlockSpec((NC, rblk, d), lambda i: (0, i, 0)),
            pl.BlockSpec((rblk, ntiles), lambda i: (i, 0)),
            pl.BlockSpec((1, d), lambda i: (0, 0)),
        ],
        out_specs=[
            pl.BlockSpec((rblk, d), lambda i: (i, 0)),
            pl.BlockSpec((1, d), lambda i: (0, 0)),
            pl.BlockSpec((1, d), lambda i: (0, 0)),
        ],
        out_shape=[
            jax.ShapeDtypeStruct((n, d), jnp.float32),
            jax.ShapeDtypeStruct((1, d), jnp.float32),
            jax.ShapeDtypeStruct((1, d), jnp.float32),
        ],
    )(po, denp.reshape(ntiles, n).T, bias.reshape(1, d))

    out = pl.pallas_call(
        functools.partial(_bn_kernel, float(n)),
        grid=(nblocks,),
        in_specs=[
            pl.BlockSpec((rblk, d), lambda i: (i, 0)),
            pl.BlockSpec((1, d), lambda i: (0, 0)),
            pl.BlockSpec((1, d), lambda i: (0, 0)),
            pl.BlockSpec((1, d), lambda i: (0, 0)),
            pl.BlockSpec((1, d), lambda i: (0, 0)),
        ],
        out_specs=pl.BlockSpec((rblk, d), lambda i: (i, 0)),
        out_shape=jax.ShapeDtypeStruct((n, d), jnp.float32),
    )(agg, colsum, colsq, bn_weight.reshape(1, d), bn_bias.reshape(1, d))
    return out


# speedup vs baseline: 1.2437x; 1.2437x over previous
import functools

import jax
import jax.numpy as jnp
from jax import lax
from jax.experimental import pallas as pl
from jax.experimental.pallas import tpu as pltpu
from jax.experimental.pallas import tpu_sc as plsc

NC = 2
NS = 16
LANES = 16
CHUNK = 48
NBUF = 2


def _matmul_kernel(x_ref, w_ref, o_ref):
    o_ref[...] = lax.dot_general(
        x_ref[...], w_ref[...], (((1,), (1,)), ((), ())),
        preferred_element_type=jnp.float32)


def _build_edge_kernel(n, d, etot, ept):
    nchunks = ept // CHUNK
    rows_per_tile = ((n + NS * 128 - 1) // (NS * 128)) * 128
    npad = rows_per_tile * NS
    zreps = rows_per_tile // CHUNK
    ngroups = CHUNK // LANES

    mesh = plsc.VectorSubcoreMesh(
        core_axis_name="c", subcore_axis_name="s",
        num_cores=NC, num_subcores=NS)

    @functools.partial(
        pl.kernel,
        out_type=(
            jax.ShapeDtypeStruct((NC, npad, d), jnp.float32),
            jax.ShapeDtypeStruct((NC * NS, 1, n), jnp.float32),
        ),
        mesh=mesh,
        compiler_params=pltpu.CompilerParams(needs_layout_passes=False),
        scratch_types=(
            [pltpu.VMEM((CHUNK,), jnp.int32) for _ in range(4)]
            + [pltpu.VMEM((CHUNK,), jnp.int32) for _ in range(4)]
            + [pltpu.VMEM((CHUNK, d), jnp.float32) for _ in range(2)]
            + [pltpu.VMEM((CHUNK, d), jnp.float32) for _ in range(2)]
            + [pltpu.VMEM((CHUNK, d), jnp.float32) for _ in range(2)]
            + [
                pltpu.VMEM((n,), jnp.float32),
                pltpu.VMEM_SHARED((npad, d), jnp.float32),
                pltpu.SemaphoreType.DMA,
                pltpu.SemaphoreType.DMA,
                pltpu.SemaphoreType.DMA,
                pltpu.SemaphoreType.DMA,
                pltpu.SemaphoreType.DMA,
                pltpu.SemaphoreType.DMA,
                pltpu.SemaphoreType.DMA,
                pltpu.SemaphoreType.DMA,
            ]
        ),
    )
    def edge_kernel(emb_hbm, h_hbm, src_hbm, dst_hbm, po_hbm, den_hbm,
                    is0, is1, is2, is3, id0, id1, id2, id3,
                    es0, es1, ed0, ed1, hs0, hs1, den_t, acc_sh,
                    gsem0, gsem1, isem0, isem1, isem2, isem3, ssem0, ssem1):
        idx_s = [is0, is1, is2, is3]
        idx_d = [id0, id1, id2, id3]
        es2 = [es0, es1]
        ed2 = [ed0, ed1]
        hs2 = [hs0, hs1]
        gsem = [gsem0, gsem1]
        isem = [isem0, isem1, isem2, isem3]
        ssem = [ssem0, ssem1]
        cid = lax.axis_index("c")
        sid = lax.axis_index("s")
        wid = cid * NS + sid
        iota = lax.broadcasted_iota(jnp.int32, (LANES,), 0)
        zeros16 = jnp.zeros((LANES,), jnp.float32)
        ebase = wid * ept

        def zden(i, carry):
            den_t[pl.ds(i * LANES, LANES)] = zeros16
            return carry
        lax.fori_loop(0, n // LANES, zden, 0)

        def zz(i, carry):
            for j in range(d // LANES):
                hs0[i, pl.ds(j * LANES, LANES)] = zeros16
            return carry
        lax.fori_loop(0, CHUNK, zz, 0)
        row0 = sid * rows_per_tile
        left = rows_per_tile
        while left > 0:
            rr = min(left, CHUNK)
            pltpu.sync_copy(hs0.at[pl.ds(0, rr)],
                            acc_sh.at[pl.ds(row0 + rows_per_tile - left, rr)])
            left -= rr

        def fetch_idx(k, b4):
            base = ebase + k * CHUNK
            pltpu.sync_copy(src_hbm.at[pl.ds(base, CHUNK)], idx_s[b4])
            pltpu.sync_copy(dst_hbm.at[pl.ds(base, CHUNK)], idx_d[b4])

        def fetch_idx_async(k, b4):
            base = ebase + k * CHUNK
            pltpu.async_copy(src_hbm.at[pl.ds(base, CHUNK)], idx_s[b4],
                             isem[b4])
            pltpu.async_copy(dst_hbm.at[pl.ds(base, CHUNK)], idx_d[b4],
                             isem[b4])

        def wait_idx(b4):
            for _ in range(2):
                pltpu.make_async_copy(src_hbm.at[pl.ds(0, CHUNK)],
                                      idx_s[b4], isem[b4]).wait()

        def issue_embs(b4, b2):
            pltpu.async_copy(emb_hbm.at[idx_s[b4]], es2[b2], gsem[b2])
            pltpu.async_copy(emb_hbm.at[idx_d[b4]], ed2[b2], gsem[b2])

        def issue_h(b4, b2):
            pltpu.async_copy(h_hbm.at[idx_s[b4]], hs2[b2], gsem[b2])

        def drain(b2, m=3):
            for _ in range(m):
                pltpu.make_async_copy(emb_hbm.at[idx_s[0]], es2[b2],
                                      gsem[b2]).wait()

        def wait_scatter(b2):
            pltpu.make_async_copy(hs2[b2], acc_sh.at[pl.ds(0, CHUNK)],
                                  ssem[b2]).wait()

        for k in (0, 1):
            fetch_idx(k, k)
            issue_embs(k, k)
            issue_h(k, k)
        fetch_idx_async(2, 2)
        fetch_idx_async(3, 3)
        plsc.subcore_barrier()

        def chunk_body(i, carry):
            for b4 in range(4):
                b2 = b4 % 2
                k = 4 * i + b4
                base = ebase + k * CHUNK
                es, ed, hs = es2[b2], ed2[b2], hs2[b2]
                drain(b2)

                for g in range(ngroups):
                    def edge_body(kk, ewg):
                        e = g * LANES + kk
                        acc = zeros16
                        for j in range(d // LANES):
                            sv = es[e, pl.ds(j * LANES, LANES)]
                            dv = ed[e, pl.ds(j * LANES, LANES)]
                            acc = acc + (sv - dv) * dv
                        av = acc
                        for sh in (8, 4, 2, 1):
                            av = av + jnp.take(av, iota ^ sh)
                        valid = jnp.full((LANES,), base + e, jnp.int32) < etot
                        ew = jnp.where(valid, jnp.exp(av), 0.0)
                        for j in range(d // LANES):
                            sl = pl.ds(j * LANES, LANES)
                            hs[e, sl] = hs[e, sl] * ew
                        return jnp.where(iota == kk, ew, ewg)
                    ewg = lax.fori_loop(0, LANES, edge_body, zeros16)
                    dd_idx = idx_d[b4][pl.ds(g * LANES, LANES)]
                    plsc.addupdate_scatter(den_t, [dd_idx], ewg)

                wait_idx((b4 + 2) % 4)
                issue_embs((b4 + 2) % 4, b2)

                pltpu.sync_copy(hs, acc_sh.at[idx_d[b4]], add=True)
                issue_h((b4 + 2) % 4, b2)
                fetch_idx_async(jnp.minimum(k + 4, nchunks - 1), b4)
            return carry
        lax.fori_loop(0, nchunks // 4, chunk_body, 0)
        drain(0, 3)
        drain(1, 3)
        wait_idx(2)
        wait_idx(3)
        plsc.subcore_barrier()

        pltpu.sync_copy(acc_sh.at[pl.ds(row0, rows_per_tile)],
                        po_hbm.at[cid, pl.ds(row0, rows_per_tile)])
        pltpu.sync_copy(den_t, den_hbm.at[wid, 0])

    return edge_kernel


def _agg_kernel(po_ref, den_ref, bias_ref, agg_ref, sum_ref, sq_ref):
    i = pl.program_id(0)
    densum = jnp.sum(den_ref[...], axis=1, keepdims=True)
    agg = (po_ref[0] + po_ref[1]) / (densum + 1e-16) + bias_ref[...]
    agg_ref[...] = agg

    @pl.when(i == 0)
    def _():
        sum_ref[...] = jnp.zeros_like(sum_ref)
        sq_ref[...] = jnp.zeros_like(sq_ref)
    sum_ref[...] += jnp.sum(agg, axis=0, keepdims=True)
    sq_ref[...] += jnp.sum(agg * agg, axis=0, keepdims=True)


def _bn_kernel(n, agg_ref, sum_ref, sq_ref, bnw_ref, bnb_ref, o_ref):
    mean = sum_ref[...] / n
    var = sq_ref[...] / n - mean * mean
    y = (agg_ref[...] - mean) * lax.rsqrt(var + 1e-5) * bnw_ref[...]
    o_ref[...] = jnp.maximum(y + bnb_ref[...], 0.0)


def kernel(x, embedding, W, bias, bn_weight, bn_bias, edge_index):
    n, d_in = x.shape
    d = embedding.shape[1]
    e = edge_index.shape[1]
    etot = e + n
    ntiles = NC * NS
    step = 4 * CHUNK
    ept = ((etot + ntiles * step - 1) // (ntiles * step)) * step
    epad = ept * ntiles

    loop = jnp.arange(n, dtype=jnp.int32)
    src = jnp.concatenate([edge_index[0].astype(jnp.int32), loop,
                           jnp.zeros((epad - etot,), jnp.int32)])
    dst = jnp.concatenate([edge_index[1].astype(jnp.int32), loop,
                           jnp.zeros((epad - etot,), jnp.int32)])

    rblk = 2000
    nblocks = n // rblk
    h = pl.pallas_call(
        _matmul_kernel,
        grid=(nblocks,),
        in_specs=[
            pl.BlockSpec((rblk, d_in), lambda i: (i, 0)),
            pl.BlockSpec((d, d_in), lambda i: (0, 0)),
        ],
        out_specs=pl.BlockSpec((rblk, d), lambda i: (i, 0)),
        out_shape=jax.ShapeDtypeStruct((n, d), jnp.float32),
    )(x, W)

    edge_kernel = _build_edge_kernel(n, d, etot, ept)
    po, denp = edge_kernel(embedding, h, src, dst)

    agg, colsum, colsq = pl.pallas_call(
        _agg_kernel,
        grid=(nblocks,),
        in_specs=[
            pl.BlockSpec((NC, rblk, d), lambda i: (0, i, 0)),
            pl.BlockSpec((rblk, ntiles), lambda i: (i, 0)),
            pl.BlockSpec((1, d), lambda i: (0, 0)),
        ],
        out_specs=[
            pl.BlockSpec((rblk, d), lambda i: (i, 0)),
            pl.BlockSpec((1, d), lambda i: (0, 0)),
            pl.BlockSpec((1, d), lambda i: (0, 0)),
        ],
        out_shape=[
            jax.ShapeDtypeStruct((n, d), jnp.float32),
            jax.ShapeDtypeStruct((1, d), jnp.float32),
            jax.ShapeDtypeStruct((1, d), jnp.float32),
        ],
    )(po, denp.reshape(ntiles, n).T, bias.reshape(1, d))

    out = pl.pallas_call(
        functools.partial(_bn_kernel, float(n)),
        grid=(nblocks,),
        in_specs=[
            pl.BlockSpec((rblk, d), lambda i: (i, 0)),
            pl.BlockSpec((1, d), lambda i: (0, 0)),
            pl.BlockSpec((1, d), lambda i: (0, 0)),
            pl.BlockSpec((1, d), lambda i: (0, 0)),
            pl.BlockSpec((1, d), lambda i: (0, 0)),
        ],
        out_specs=pl.BlockSpec((rblk, d), lambda i: (i, 0)),
        out_shape=jax.ShapeDtypeStruct((n, d), jnp.float32),
    )(agg, colsum, colsq, bn_weight.reshape(1, d), bn_bias.reshape(1, d))
    return out
